# baseline (speedup 1.0000x reference)
import functools

import jax
import jax.numpy as jnp
from jax import lax
from jax.experimental import pallas as pl
from jax.experimental.pallas import tpu as pltpu
from jax.experimental.pallas import tpu_sc as plsc

N = 10000
E = 320000
DIN = 128
HID = 64
EMB = 32
EPS = 1e-5

NC = 2
NS = 16
NW = NC * NS
ETOT = E + N
K = 82
EP = NW * K * 128
NPAD = 10240
RPT = NPAD // NS
BR = 1000


def _sc_mesh():
    return plsc.VectorSubcoreMesh(core_axis_name="c", subcore_axis_name="s")


_SC_PARAMS = pltpu.CompilerParams(use_tc_tiling_on_sc=False)


def _deg_call(dst3, ones, zeros):
    @functools.partial(
        pl.kernel,
        mesh=_sc_mesh(),
        compiler_params=_SC_PARAMS,
        out_type=jax.ShapeDtypeStruct((NC, NPAD, 16), jnp.float32),
        scratch_types=[
            pltpu.VMEM((K, 128), jnp.int32),
            pltpu.VMEM((128, 16), jnp.float32),
            pltpu.VMEM_SHARED((NPAD, 16), jnp.float32),
            pltpu.SemaphoreType.DMA,
        ],
    )
    def deg_kernel(dst_hbm, ones_hbm, zeros_hbm, out_hbm, idx_v, ones_v,
                   acc_sh, sem):
        cid = lax.axis_index("c")
        sid = lax.axis_index("s")
        wid = cid * NS + sid
        pltpu.sync_copy(dst_hbm.at[wid], idx_v)
        pltpu.sync_copy(ones_hbm, ones_v)
        pltpu.sync_copy(zeros_hbm, acc_sh.at[pl.ds(sid * RPT, RPT)])
        plsc.subcore_barrier()

        @pl.loop(0, K)
        def _(j):
            pltpu.async_copy(ones_v, acc_sh.at[idx_v.at[j]], sem, add=True)

        @pl.loop(0, K)
        def _(j):
            pltpu.make_async_copy(ones_v, acc_sh.at[idx_v.at[0]], sem).wait()

        plsc.subcore_barrier()
        pltpu.sync_copy(acc_sh.at[pl.ds(sid * RPT, RPT)],
                        out_hbm.at[cid, pl.ds(sid * RPT, RPT)])

    return deg_kernel(dst3, ones, zeros)


def _gs_call(src3, dst3, table, zeros, feat):
    @functools.partial(
        pl.kernel,
        mesh=_sc_mesh(),
        compiler_params=_SC_PARAMS,
        out_type=jax.ShapeDtypeStruct((NC, NPAD, feat), jnp.float32),
        scratch_types=[
            pltpu.VMEM((K, 128), jnp.int32),
            pltpu.VMEM((K, 128), jnp.int32),
            pltpu.VMEM((128, feat), jnp.float32),
            pltpu.VMEM((128, feat), jnp.float32),
            pltpu.VMEM_SHARED((NPAD, feat), jnp.float32),
            pltpu.SemaphoreType.DMA,
            pltpu.SemaphoreType.DMA,
            pltpu.SemaphoreType.DMA,
            pltpu.SemaphoreType.DMA,
        ],
    )
    def gs_kernel(src_hbm, dst_hbm, table_hbm, zeros_hbm, out_hbm,
                  src_v, dst_v, rows_a, rows_b, acc_sh, gsa, gsb, ssa, ssb):
        cid = lax.axis_index("c")
        sid = lax.axis_index("s")
        wid = cid * NS + sid
        pltpu.sync_copy(src_hbm.at[wid], src_v)
        pltpu.sync_copy(dst_hbm.at[wid], dst_v)
        pltpu.sync_copy(zeros_hbm, acc_sh.at[pl.ds(sid * RPT, RPT)])
        plsc.subcore_barrier()

        def gstart(j, buf, sem):
            pltpu.async_copy(table_hbm.at[src_v.at[j]], buf, sem)

        def gwait(buf, sem):
            pltpu.make_async_copy(table_hbm.at[src_v.at[0]], buf, sem).wait()

        def sstart(j, buf, sem):
            pltpu.async_copy(buf, acc_sh.at[dst_v.at[j]], sem, add=True)

        def swait(buf, sem):
            pltpu.make_async_copy(buf, acc_sh.at[dst_v.at[0]], sem).wait()

        gstart(0, rows_a, gsa)

        @pl.loop(0, K // 2)
        def _(t):
            j0 = 2 * t
            j1 = j0 + 1
            gwait(rows_a, gsa)

            @pl.when(t > 0)
            def _():
                swait(rows_b, ssb)

            gstart(j1, rows_b, gsb)
            sstart(j0, rows_a, ssa)
            gwait(rows_b, gsb)
            swait(rows_a, ssa)

            @pl.when(j1 + 1 < K)
            def _():
                gstart(j1 + 1, rows_a, gsa)

            sstart(j1, rows_b, ssb)

        swait(rows_b, ssb)
        plsc.subcore_barrier()
        pltpu.sync_copy(acc_sh.at[pl.ds(sid * RPT, RPT)],
                        out_hbm.at[cid, pl.ds(sid * RPT, RPT)])

    return gs_kernel(src3, dst3, table, zeros)


def _dis_of(degp_ref):
    deg = degp_ref[0, :, 0] + degp_ref[1, :, 0]
    return lax.rsqrt(deg)[:, None]


def _mm1_body(x_ref, w_ref, degp_ref, o_ref):
    o_ref[...] = jnp.dot(x_ref[...], w_ref[...],
                         preferred_element_type=jnp.float32,
                         precision=lax.Precision.HIGHEST) * _dis_of(degp_ref)


def _mm1_call(x, W1, degp):
    return pl.pallas_call(
        _mm1_body,
        grid=(N // BR,),
        in_specs=[
            pl.BlockSpec((BR, DIN), lambda i: (i, 0)),
            pl.BlockSpec((DIN, HID), lambda i: (0, 0)),
            pl.BlockSpec((NC, BR, 16), lambda i: (0, i, 0)),
        ],
        out_specs=pl.BlockSpec((BR, HID), lambda i: (i, 0)),
        out_shape=jax.ShapeDtypeStruct((N, HID), jnp.float32),
    )(x, W1, degp)


def _fuse_body(accp_ref, degp_ref, b_ref, g_ref, be_ref, rm_ref, rv_ref,
               w_ref, o_ref):
    dis = _dis_of(degp_ref)
    conv = (accp_ref[0] + accp_ref[1]) * dis + b_ref[...]
    h = (conv - rm_ref[...]) * lax.rsqrt(rv_ref[...] + EPS) * g_ref[...] \
        + be_ref[...]
    h = jnp.maximum(h, 0.0)
    o_ref[...] = jnp.dot(h, w_ref[...],
                         preferred_element_type=jnp.float32,
                         precision=lax.Precision.HIGHEST) * dis


def _fuse_call(accp, degp, b, g, be, rm, rv, W, fout):
    vec = lambda a: a.reshape(1, HID)
    return pl.pallas_call(
        _fuse_body,
        grid=(N // BR,),
        in_specs=[
            pl.BlockSpec((NC, BR, HID), lambda i: (0, i, 0)),
            pl.BlockSpec((NC, BR, 16), lambda i: (0, i, 0)),
            pl.BlockSpec((1, HID), lambda i: (0, 0)),
            pl.BlockSpec((1, HID), lambda i: (0, 0)),
            pl.BlockSpec((1, HID), lambda i: (0, 0)),
            pl.BlockSpec((1, HID), lambda i: (0, 0)),
            pl.BlockSpec((1, HID), lambda i: (0, 0)),
            pl.BlockSpec((HID, fout), lambda i: (0, 0)),
        ],
        out_specs=pl.BlockSpec((BR, fout), lambda i: (i, 0)),
        out_shape=jax.ShapeDtypeStruct((N, fout), jnp.float32),
    )(accp, degp, vec(b), vec(g), vec(be), vec(rm), vec(rv), W)


def _final_body(accp_ref, degp_ref, b_ref, o_ref):
    o_ref[...] = (accp_ref[0] + accp_ref[1]) * _dis_of(degp_ref) + b_ref[...]


def _final_call(accp, degp, b3):
    return pl.pallas_call(
        _final_body,
        grid=(N // BR,),
        in_specs=[
            pl.BlockSpec((NC, BR, EMB), lambda i: (0, i, 0)),
            pl.BlockSpec((NC, BR, 16), lambda i: (0, i, 0)),
            pl.BlockSpec((1, EMB), lambda i: (0, 0)),
        ],
        out_specs=pl.BlockSpec((BR, EMB), lambda i: (i, 0)),
        out_shape=jax.ShapeDtypeStruct((N, EMB), jnp.float32),
    )(accp, degp, b3.reshape(1, EMB))


def kernel(x, edge_index, W1, b1, g1, be1, rm1, rv1,
           W2, b2, g2, be2, rm2, rv2, W3, b3):
    loops = jnp.arange(N, dtype=jnp.int32)
    pad = EP - ETOT
    pad_iota = jnp.arange(pad, dtype=jnp.int32)
    src = jnp.concatenate([edge_index[0], loops, pad_iota % N])
    dst = jnp.concatenate([edge_index[1], loops, N + pad_iota % (NPAD - N)])
    src3 = src.reshape(NW, K, 128)
    dst3 = dst.reshape(NW, K, 128)

    ones16 = jnp.ones((128, 16), jnp.float32)
    zeros16 = jnp.zeros((RPT, 16), jnp.float32)
    zerosH = jnp.zeros((RPT, HID), jnp.float32)
    zerosE = jnp.zeros((RPT, EMB), jnp.float32)

    degp = _deg_call(dst3, ones16, zeros16)
    y1 = _mm1_call(x, W1, degp)
    acc1 = _gs_call(src3, dst3, y1, zerosH, HID)
    y2 = _fuse_call(acc1, degp, b1, g1, be1, rm1, rv1, W2, HID)
    acc2 = _gs_call(src3, dst3, y2, zerosH, HID)
    y3 = _fuse_call(acc2, degp, b2, g2, be2, rm2, rv2, W3, EMB)
    acc3 = _gs_call(src3, dst3, y3, zerosE, EMB)
    return _final_call(acc3, degp, b3)


# speedup vs baseline: 1.2314x; 1.2314x over previous
import functools

import jax
import jax.numpy as jnp
from jax import lax
from jax.experimental import pallas as pl
from jax.experimental.pallas import tpu as pltpu
from jax.experimental.pallas import tpu_sc as plsc

N = 10000
E = 320000
DIN = 128
HID = 64
EMB = 32
EPS = 1e-5

NC = 2
NS = 16
NW = NC * NS
ETOT = E + N
G = 3
K4 = 28
K = G * K4
EP = NW * K * 128
NPAD = 10240
RPT = NPAD // NS
BR = 1000


def _sc_mesh():
    return plsc.VectorSubcoreMesh(core_axis_name="c", subcore_axis_name="s")


_SC_PARAMS = pltpu.CompilerParams(use_tc_tiling_on_sc=False)


def _deg_call(dst3, ones, zeros):
    @functools.partial(
        pl.kernel,
        mesh=_sc_mesh(),
        compiler_params=_SC_PARAMS,
        out_type=jax.ShapeDtypeStruct((NC, NPAD, 16), jnp.float32),
        scratch_types=[
            pltpu.VMEM((K4, G * 128), jnp.int32),
            pltpu.VMEM((G * 128, 16), jnp.float32),
            pltpu.VMEM_SHARED((NPAD, 16), jnp.float32),
            pltpu.SemaphoreType.DMA,
        ],
    )
    def deg_kernel(dst_hbm, ones_hbm, zeros_hbm, out_hbm, idx_v, ones_v,
                   acc_sh, sem):
        cid = lax.axis_index("c")
        sid = lax.axis_index("s")
        wid = cid * NS + sid
        pltpu.sync_copy(dst_hbm.at[wid], idx_v)
        pltpu.sync_copy(ones_hbm, ones_v)
        pltpu.sync_copy(zeros_hbm, acc_sh.at[pl.ds(sid * RPT, RPT)])
        plsc.subcore_barrier()

        @pl.loop(0, K4)
        def _(j):
            pltpu.async_copy(ones_v, acc_sh.at[idx_v.at[j]], sem, add=True)

        @pl.loop(0, K4)
        def _(j):
            pltpu.make_async_copy(ones_v, acc_sh.at[idx_v.at[0]], sem).wait()

        plsc.subcore_barrier()
        pltpu.sync_copy(acc_sh.at[pl.ds(sid * RPT, RPT)],
                        out_hbm.at[cid, pl.ds(sid * RPT, RPT)])

    return deg_kernel(dst3, ones, zeros)


def _gs_call(src3, dst3, table, zeros, feat):
    @functools.partial(
        pl.kernel,
        mesh=_sc_mesh(),
        compiler_params=_SC_PARAMS,
        out_type=jax.ShapeDtypeStruct((NC, NPAD, feat), jnp.float32),
        scratch_types=[
            pltpu.VMEM((K4, G * 128), jnp.int32),
            pltpu.VMEM((K4, G * 128), jnp.int32),
            pltpu.VMEM((G * 128, feat), jnp.float32),
            pltpu.VMEM((G * 128, feat), jnp.float32),
            pltpu.VMEM_SHARED((NPAD, feat), jnp.float32),
            pltpu.SemaphoreType.DMA,
            pltpu.SemaphoreType.DMA,
            pltpu.SemaphoreType.DMA,
            pltpu.SemaphoreType.DMA,
        ],
    )
    def gs_kernel(src_hbm, dst_hbm, table_hbm, zeros_hbm, out_hbm,
                  src_v, dst_v, rows_a, rows_b, acc_sh, gsa, gsb, ssa, ssb):
        cid = lax.axis_index("c")
        sid = lax.axis_index("s")
        wid = cid * NS + sid
        pltpu.sync_copy(src_hbm.at[wid], src_v)
        pltpu.sync_copy(dst_hbm.at[wid], dst_v)
        pltpu.sync_copy(zeros_hbm, acc_sh.at[pl.ds(sid * RPT, RPT)])
        plsc.subcore_barrier()

        def gstart(j, buf, sem):
            pltpu.async_copy(table_hbm.at[src_v.at[j]], buf, sem)

        def gwait(buf, sem):
            pltpu.make_async_copy(table_hbm.at[src_v.at[0]], buf, sem).wait()

        def sstart(j, buf, sem):
            pltpu.async_copy(buf, acc_sh.at[dst_v.at[j]], sem, add=True)

        def swait(buf, sem):
            pltpu.make_async_copy(buf, acc_sh.at[dst_v.at[0]], sem).wait()

        gstart(0, rows_a, gsa)

        @pl.loop(0, K4 // 2)
        def _(t):
            j0 = 2 * t
            j1 = j0 + 1
            gwait(rows_a, gsa)

            @pl.when(t > 0)
            def _():
                swait(rows_b, ssb)

            gstart(j1, rows_b, gsb)
            sstart(j0, rows_a, ssa)
            gwait(rows_b, gsb)
            swait(rows_a, ssa)

            @pl.when(j1 + 1 < K4)
            def _():
                gstart(j1 + 1, rows_a, gsa)

            sstart(j1, rows_b, ssb)

        swait(rows_b, ssb)
        plsc.subcore_barrier()
        pltpu.sync_copy(acc_sh.at[pl.ds(sid * RPT, RPT)],
                        out_hbm.at[cid, pl.ds(sid * RPT, RPT)])

    return gs_kernel(src3, dst3, table, zeros)


def _dis_of(degp_ref):
    deg = degp_ref[0, :, 0] + degp_ref[1, :, 0]
    return lax.rsqrt(deg)[:, None]


def _mm1_body(x_ref, w_ref, degp_ref, o_ref):
    o_ref[...] = jnp.dot(x_ref[...], w_ref[...],
                         preferred_element_type=jnp.float32,
                         precision=lax.Precision.HIGHEST) * _dis_of(degp_ref)


def _mm1_call(x, W1, degp):
    return pl.pallas_call(
        _mm1_body,
        grid=(N // BR,),
        in_specs=[
            pl.BlockSpec((BR, DIN), lambda i: (i, 0)),
            pl.BlockSpec((DIN, HID), lambda i: (0, 0)),
            pl.BlockSpec((NC, BR, 16), lambda i: (0, i, 0)),
        ],
        out_specs=pl.BlockSpec((BR, HID), lambda i: (i, 0)),
        out_shape=jax.ShapeDtypeStruct((N, HID), jnp.float32),
    )(x, W1, degp)


def _fuse_body(accp_ref, degp_ref, b_ref, g_ref, be_ref, rm_ref, rv_ref,
               w_ref, o_ref):
    dis = _dis_of(degp_ref)
    conv = (accp_ref[0] + accp_ref[1]) * dis + b_ref[...]
    h = (conv - rm_ref[...]) * lax.rsqrt(rv_ref[...] + EPS) * g_ref[...] \
        + be_ref[...]
    h = jnp.maximum(h, 0.0)
    o_ref[...] = jnp.dot(h, w_ref[...],
                         preferred_element_type=jnp.float32,
                         precision=lax.Precision.HIGHEST) * dis


def _fuse_call(accp, degp, b, g, be, rm, rv, W, fout):
    vec = lambda a: a.reshape(1, HID)
    return pl.pallas_call(
        _fuse_body,
        grid=(N // BR,),
        in_specs=[
            pl.BlockSpec((NC, BR, HID), lambda i: (0, i, 0)),
            pl.BlockSpec((NC, BR, 16), lambda i: (0, i, 0)),
            pl.BlockSpec((1, HID), lambda i: (0, 0)),
            pl.BlockSpec((1, HID), lambda i: (0, 0)),
            pl.BlockSpec((1, HID), lambda i: (0, 0)),
            pl.BlockSpec((1, HID), lambda i: (0, 0)),
            pl.BlockSpec((1, HID), lambda i: (0, 0)),
            pl.BlockSpec((HID, fout), lambda i: (0, 0)),
        ],
        out_specs=pl.BlockSpec((BR, fout), lambda i: (i, 0)),
        out_shape=jax.ShapeDtypeStruct((N, fout), jnp.float32),
    )(accp, degp, vec(b), vec(g), vec(be), vec(rm), vec(rv), W)


def _final_body(accp_ref, degp_ref, b_ref, o_ref):
    o_ref[...] = (accp_ref[0] + accp_ref[1]) * _dis_of(degp_ref) + b_ref[...]


def _final_call(accp, degp, b3):
    return pl.pallas_call(
        _final_body,
        grid=(N // BR,),
        in_specs=[
            pl.BlockSpec((NC, BR, EMB), lambda i: (0, i, 0)),
            pl.BlockSpec((NC, BR, 16), lambda i: (0, i, 0)),
            pl.BlockSpec((1, EMB), lambda i: (0, 0)),
        ],
        out_specs=pl.BlockSpec((BR, EMB), lambda i: (i, 0)),
        out_shape=jax.ShapeDtypeStruct((N, EMB), jnp.float32),
    )(accp, degp, b3.reshape(1, EMB))


def kernel(x, edge_index, W1, b1, g1, be1, rm1, rv1,
           W2, b2, g2, be2, rm2, rv2, W3, b3):
    loops = jnp.arange(N, dtype=jnp.int32)
    pad = EP - ETOT
    pad_iota = jnp.arange(pad, dtype=jnp.int32)
    src = jnp.concatenate([edge_index[0], loops, pad_iota % N])
    dst = jnp.concatenate([edge_index[1], loops, N + pad_iota % (NPAD - N)])
    src3 = src.reshape(NW, K4, G * 128)
    dst3 = dst.reshape(NW, K4, G * 128)

    ones16 = jnp.ones((G * 128, 16), jnp.float32)
    zeros16 = jnp.zeros((RPT, 16), jnp.float32)
    zerosH = jnp.zeros((RPT, HID), jnp.float32)
    zerosE = jnp.zeros((RPT, EMB), jnp.float32)

    degp = _deg_call(dst3, ones16, zeros16)
    y1 = _mm1_call(x, W1, degp)
    acc1 = _gs_call(src3, dst3, y1, zerosH, HID)
    y2 = _fuse_call(acc1, degp, b1, g1, be1, rm1, rv1, W2, HID)
    acc2 = _gs_call(src3, dst3, y2, zerosH, HID)
    y3 = _fuse_call(acc2, degp, b2, g2, be2, rm2, rv2, W3, EMB)
    acc3 = _gs_call(src3, dst3, y3, zerosE, EMB)
    return _final_call(acc3, degp, b3)
